# baseline (speedup 1.0000x reference)
import functools

import jax
import jax.numpy as jnp
from jax import lax
from jax.experimental import pallas as pl
from jax.experimental.pallas import tpu as pltpu
from jax.experimental.pallas import tpu_sc as plsc


def _sc_gather(emb, idx, cpw, nw, nc):
    d = emb.shape[1]
    n_rows = idx.shape[0]
    rows_per_w = cpw * 128
    mesh = plsc.VectorSubcoreMesh(core_axis_name="c", subcore_axis_name="s")

    @functools.partial(
        pl.kernel,
        mesh=mesh,
        out_type=jax.ShapeDtypeStruct((n_rows, d), jnp.float32),
        scratch_types=[
            pltpu.VMEM((rows_per_w,), jnp.int32),
            pltpu.VMEM((rows_per_w, d), jnp.float32),
            pltpu.SemaphoreType.DMA,
        ],
        compiler_params=pltpu.CompilerParams(use_tc_tiling_on_sc=False),
    )
    def gather_kernel(emb_hbm, idx_hbm, out_hbm, idx_v, rows_v, sem):
        wid = lax.axis_index("s") * nc + lax.axis_index("c")
        base = wid * rows_per_w
        pltpu.sync_copy(idx_hbm.at[pl.ds(base, rows_per_w)], idx_v)
        copies = []
        for j in range(cpw):
            copies.append(
                pltpu.async_copy(
                    emb_hbm.at[idx_v.at[pl.ds(j * 128, 128)]],
                    rows_v.at[pl.ds(j * 128, 128)],
                    sem,
                )
            )
        for c in copies:
            c.wait()
        pltpu.sync_copy(rows_v, out_hbm.at[pl.ds(base, rows_per_w)])

    return gather_kernel(emb, idx)


def _tc_mm1_t(flat, w1, b1):
    b, f = flat.shape
    h = w1.shape[0]

    def body(flat_ref, w1_ref, b1_ref, ht_ref):
        acc = lax.dot_general(
            w1_ref[...], flat_ref[...], (((1,), (1,)), ((), ())),
            preferred_element_type=jnp.float32)
        ht_ref[...] = jnp.maximum(acc + b1_ref[...][:, None], 0.0)

    return pl.pallas_call(
        body,
        out_shape=jax.ShapeDtypeStruct((h, b), jnp.float32),
    )(flat, w1, b1)


def _tc_mm2_t(ht, w2t, b2, bv=4096):
    h, b = ht.shape
    v = w2t.shape[1]
    ns = pl.cdiv(v, bv)

    def body(ht_ref, w2t_ref, b2_ref, out_ref):
        acc = lax.dot_general(
            w2t_ref[...], ht_ref[...], (((0,), (0,)), ((), ())),
            preferred_element_type=jnp.float32)
        out_ref[...] = acc + b2_ref[...][:, None]

    return pl.pallas_call(
        body,
        grid=(ns,),
        in_specs=[
            pl.BlockSpec((h, b), lambda j: (0, 0)),
            pl.BlockSpec((h, bv), lambda j: (0, j)),
            pl.BlockSpec((bv,), lambda j: (j,)),
        ],
        out_specs=pl.BlockSpec((bv, b), lambda j: (j, 0)),
        out_shape=jax.ShapeDtypeStruct((v, b), jnp.float32),
        compiler_params=pltpu.CompilerParams(
            dimension_semantics=("parallel",)),
    )(ht, w2t, b2)


def kernel(x, emb, W1, b1, W2, b2):
    batch, ctx = x.shape
    d = emb.shape[1]
    n = batch * ctx

    info = plsc.get_sparse_core_info()
    nc, ns = info.num_cores, info.num_subcores
    nw = nc * ns
    assert n % (128 * nw) == 0
    cpw = n // (128 * nw)

    idx = x.reshape(n).astype(jnp.int32)
    rows = _sc_gather(emb, idx, cpw, nw, nc)
    flat = rows.reshape(batch, ctx * d)
    ht = _tc_mm1_t(flat, W1, b1)
    out_t = _tc_mm2_t(ht, W2.T, b2)
    return out_t.T


# speedup vs baseline: 1.0123x; 1.0123x over previous
import functools

import jax
import jax.numpy as jnp
from jax import lax
from jax.experimental import pallas as pl
from jax.experimental.pallas import tpu as pltpu
from jax.experimental.pallas import tpu_sc as plsc


def _sc_gather(emb, idx, cpw, nw, nc):
    d = emb.shape[1]
    n_rows = idx.shape[0]
    rows_per_w = cpw * 128
    mesh = plsc.VectorSubcoreMesh(core_axis_name="c", subcore_axis_name="s")

    @functools.partial(
        pl.kernel,
        mesh=mesh,
        out_type=jax.ShapeDtypeStruct((n_rows, d), jnp.float32),
        scratch_types=[
            pltpu.VMEM((rows_per_w,), jnp.int32),
            pltpu.VMEM((rows_per_w, d), jnp.float32),
            pltpu.SemaphoreType.DMA,
        ],
        compiler_params=pltpu.CompilerParams(use_tc_tiling_on_sc=False),
    )
    def gather_kernel(emb_hbm, idx_hbm, out_hbm, idx_v, rows_v, sem):
        wid = lax.axis_index("s") * nc + lax.axis_index("c")
        base = wid * rows_per_w
        pltpu.sync_copy(idx_hbm.at[pl.ds(base, rows_per_w)], idx_v)
        copies = []
        for j in range(cpw):
            copies.append(
                pltpu.async_copy(
                    emb_hbm.at[idx_v.at[pl.ds(j * 128, 128)]],
                    rows_v.at[pl.ds(j * 128, 128)],
                    sem,
                )
            )
        for c in copies:
            c.wait()
        pltpu.sync_copy(rows_v, out_hbm.at[pl.ds(base, rows_per_w)])

    return gather_kernel(emb, idx)


def _tc_mlp_t(flat, w1, b1, w2t, b2, bv=4096):
    b, f = flat.shape
    h = w1.shape[0]
    v = w2t.shape[1]
    ns = pl.cdiv(v, bv)

    def body(flat_ref, w1_ref, b1_ref, w2t_ref, b2_ref, out_ref, ht_ref):
        @pl.when(pl.program_id(0) == 0)
        def _():
            acc = lax.dot_general(
                w1_ref[...], flat_ref[...], (((1,), (1,)), ((), ())),
                preferred_element_type=jnp.float32)
            ht_ref[...] = jnp.maximum(
                acc + b1_ref[...][:, None], 0.0).astype(jnp.bfloat16)

        acc = lax.dot_general(
            w2t_ref[...].astype(jnp.bfloat16), ht_ref[...],
            (((0,), (0,)), ((), ())),
            preferred_element_type=jnp.float32)
        out_ref[...] = acc + b2_ref[...][:, None]

    return pl.pallas_call(
        body,
        grid=(ns,),
        in_specs=[
            pl.BlockSpec((b, f), lambda j: (0, 0)),
            pl.BlockSpec((h, f), lambda j: (0, 0)),
            pl.BlockSpec((h,), lambda j: (0,)),
            pl.BlockSpec((h, bv), lambda j: (0, j)),
            pl.BlockSpec((bv,), lambda j: (j,)),
        ],
        out_specs=pl.BlockSpec((bv, b), lambda j: (j, 0)),
        out_shape=jax.ShapeDtypeStruct((v, b), jnp.float32),
        scratch_shapes=[pltpu.VMEM((h, b), jnp.bfloat16)],
    )(flat, w1, b1, w2t, b2)


def kernel(x, emb, W1, b1, W2, b2):
    batch, ctx = x.shape
    d = emb.shape[1]
    n = batch * ctx

    info = plsc.get_sparse_core_info()
    nc, ns = info.num_cores, info.num_subcores
    nw = nc * ns
    assert n % (128 * nw) == 0
    cpw = n // (128 * nw)

    idx = x.reshape(n).astype(jnp.int32)
    rows = _sc_gather(emb, idx, cpw, nw, nc)
    flat = rows.reshape(batch, ctx * d)
    out_t = _tc_mlp_t(flat, W1, b1, W2.T, b2)
    return out_t.T


# speedup vs baseline: 1.0321x; 1.0195x over previous
import functools

import jax
import jax.numpy as jnp
from jax import lax
from jax.experimental import pallas as pl
from jax.experimental.pallas import tpu as pltpu
from jax.experimental.pallas import tpu_sc as plsc


def _sc_gather(emb, idx, cpw, nw, nc):
    d = emb.shape[1]
    n_rows = idx.shape[0]
    rows_per_w = cpw * 128
    mesh = plsc.VectorSubcoreMesh(core_axis_name="c", subcore_axis_name="s")

    @functools.partial(
        pl.kernel,
        mesh=mesh,
        out_type=jax.ShapeDtypeStruct((n_rows, d), jnp.float32),
        scratch_types=[
            pltpu.VMEM((rows_per_w,), jnp.int32),
            pltpu.VMEM((rows_per_w, d), jnp.float32),
            pltpu.SemaphoreType.DMA,
        ],
        compiler_params=pltpu.CompilerParams(use_tc_tiling_on_sc=False),
    )
    def gather_kernel(emb_hbm, idx_hbm, out_hbm, idx_v, rows_v, sem):
        wid = lax.axis_index("s") * nc + lax.axis_index("c")
        base = wid * rows_per_w
        pltpu.sync_copy(idx_hbm.at[pl.ds(base, rows_per_w)], idx_v)
        copies = []
        for j in range(cpw):
            copies.append(
                pltpu.async_copy(
                    emb_hbm.at[idx_v.at[pl.ds(j * 128, 128)]],
                    rows_v.at[pl.ds(j * 128, 128)],
                    sem,
                )
            )
        for c in copies:
            c.wait()
        pltpu.sync_copy(rows_v, out_hbm.at[pl.ds(base, rows_per_w)])

    return gather_kernel(emb, idx)


def _sc_gather_t(embt, xt, nw, nc):
    d, v = embt.shape
    ctx, batch = xt.shape
    assert nw == d and batch % 128 == 0
    nchunk = batch // 128
    mesh = plsc.VectorSubcoreMesh(core_axis_name="c", subcore_axis_name="s")

    @functools.partial(
        pl.kernel,
        mesh=mesh,
        out_type=jax.ShapeDtypeStruct((d, ctx, batch), jnp.float32),
        scratch_types=[
            pltpu.VMEM((ctx, batch), jnp.int32),
            pltpu.VMEM((ctx, batch), jnp.float32),
            pltpu.SemaphoreType.DMA,
        ],
        compiler_params=pltpu.CompilerParams(use_tc_tiling_on_sc=False),
    )
    def gather_kernel(embt_hbm, xt_hbm, out_hbm, idx_v, rows_v, sem):
        wid = lax.axis_index("s") * nc + lax.axis_index("c")
        pltpu.sync_copy(xt_hbm, idx_v)
        row = embt_hbm.at[wid]
        copies = []
        for c in range(ctx):
            for j in range(nchunk):
                copies.append(
                    pltpu.async_copy(
                        row.at[idx_v.at[c, pl.ds(j * 128, 128)]],
                        rows_v.at[c, pl.ds(j * 128, 128)],
                        sem,
                    )
                )
        for cp in copies:
            cp.wait()
        pltpu.sync_copy(rows_v, out_hbm.at[wid])

    return gather_kernel(embt, xt)


def _tc_mlp_t(flat_t, w1, b1, w2t, b2, bv=4096):
    f, b = flat_t.shape
    h = w1.shape[0]
    v = w2t.shape[1]
    ns = pl.cdiv(v, bv)

    def body(flat_t_ref, w1_ref, b1_ref, w2t_ref, b2_ref, out_ref, ht_ref):
        @pl.when(pl.program_id(0) == 0)
        def _():
            acc = lax.dot_general(
                w1_ref[...], flat_t_ref[...], (((1,), (0,)), ((), ())),
                preferred_element_type=jnp.float32)
            ht_ref[...] = jnp.maximum(acc + b1_ref[...][:, None], 0.0)

        acc = lax.dot_general(
            w2t_ref[...], ht_ref[...], (((0,), (0,)), ((), ())),
            preferred_element_type=jnp.float32)
        out_ref[...] = acc + b2_ref[...][:, None]

    return pl.pallas_call(
        body,
        grid=(ns,),
        in_specs=[
            pl.BlockSpec((f, b), lambda j: (0, 0)),
            pl.BlockSpec((h, f), lambda j: (0, 0)),
            pl.BlockSpec((h,), lambda j: (0,)),
            pl.BlockSpec((h, bv), lambda j: (0, j)),
            pl.BlockSpec((bv,), lambda j: (j,)),
        ],
        out_specs=pl.BlockSpec((bv, b), lambda j: (j, 0)),
        out_shape=jax.ShapeDtypeStruct((v, b), jnp.float32),
        scratch_shapes=[pltpu.VMEM((h, b), jnp.float32)],
    )(flat_t, w1, b1, w2t, b2)


def kernel(x, emb, W1, b1, W2, b2):
    batch, ctx = x.shape
    d = emb.shape[1]
    n = batch * ctx

    info = plsc.get_sparse_core_info()
    nc, ns = info.num_cores, info.num_subcores
    nw = nc * ns
    assert nw == d and batch % 128 == 0

    g = _sc_gather_t(emb.T, x.T.astype(jnp.int32), nw, nc)
    flat_t = g.reshape(d * ctx, batch)
    w1p = W1.reshape(-1, ctx, d).swapaxes(1, 2).reshape(-1, d * ctx)
    out_t = _tc_mlp_t(flat_t, w1p, b1, W2.T, b2)
    return out_t.T


# speedup vs baseline: 1.0336x; 1.0014x over previous
import functools

import jax
import jax.numpy as jnp
from jax import lax
from jax.experimental import pallas as pl
from jax.experimental.pallas import tpu as pltpu
from jax.experimental.pallas import tpu_sc as plsc


def _sc_gather_t(embt, xt, nw, nc):
    d, v = embt.shape
    ctx, batch = xt.shape
    assert nw == d and batch % 128 == 0
    nchunk = batch // 128
    mesh = plsc.VectorSubcoreMesh(core_axis_name="c", subcore_axis_name="s")

    @functools.partial(
        pl.kernel,
        mesh=mesh,
        out_type=jax.ShapeDtypeStruct((d, ctx, batch), jnp.float32),
        scratch_types=[
            pltpu.VMEM((ctx, batch), jnp.int32),
            pltpu.VMEM((ctx, batch), jnp.float32),
            pltpu.SemaphoreType.DMA,
        ],
        compiler_params=pltpu.CompilerParams(use_tc_tiling_on_sc=False),
    )
    def gather_kernel(embt_hbm, xt_hbm, out_hbm, idx_v, rows_v, sem):
        wid = lax.axis_index("s") * nc + lax.axis_index("c")
        pltpu.sync_copy(xt_hbm, idx_v)
        row = embt_hbm.at[wid]
        copies = []
        for c in range(ctx):
            for j in range(nchunk):
                copies.append(
                    pltpu.async_copy(
                        row.at[idx_v.at[c, pl.ds(j * 128, 128)]],
                        rows_v.at[c, pl.ds(j * 128, 128)],
                        sem,
                    )
                )
        for cp in copies:
            cp.wait()
        pltpu.sync_copy(rows_v, out_hbm.at[wid])

    return gather_kernel(embt, xt)


def _tc_mlp_t(flat_t, w1, b1, w2t, b2, bv=4096):
    f, b = flat_t.shape
    h = w1.shape[0]
    v = w2t.shape[1]
    ns = pl.cdiv(v, bv)

    def body(flat_t_ref, w1_ref, b1_ref, w2t_ref, b2_ref, out_ref, ht_ref):
        @pl.when(pl.program_id(0) == 0)
        def _():
            acc = lax.dot_general(
                w1_ref[...], flat_t_ref[...], (((1,), (0,)), ((), ())),
                preferred_element_type=jnp.float32)
            ht_ref[...] = jnp.maximum(acc + b1_ref[...][:, None], 0.0)

        acc = lax.dot_general(
            w2t_ref[...], ht_ref[...], (((0,), (0,)), ((), ())),
            preferred_element_type=jnp.float32)
        out_ref[...] = acc + b2_ref[...][:, None]

    return pl.pallas_call(
        body,
        grid=(ns,),
        in_specs=[
            pl.BlockSpec((f, b), lambda j: (0, 0)),
            pl.BlockSpec((h, f), lambda j: (0, 0)),
            pl.BlockSpec((h,), lambda j: (0,)),
            pl.BlockSpec((h, bv), lambda j: (0, j)),
            pl.BlockSpec((bv,), lambda j: (j,)),
        ],
        out_specs=pl.BlockSpec((bv, b), lambda j: (j, 0)),
        out_shape=jax.ShapeDtypeStruct((v, b), jnp.float32),
        scratch_shapes=[pltpu.VMEM((h, b), jnp.float32)],
    )(flat_t, w1, b1, w2t, b2)


def kernel(x, emb, W1, b1, W2, b2):
    batch, ctx = x.shape
    d = emb.shape[1]

    info = plsc.get_sparse_core_info()
    nc, ns = info.num_cores, info.num_subcores
    nw = nc * ns
    assert nw == d and batch % 128 == 0

    g = _sc_gather_t(emb.T, x.T.astype(jnp.int32), nw, nc)
    flat_t = g.reshape(d * ctx, batch)
    w1p = W1.reshape(-1, ctx, d).swapaxes(1, 2).reshape(-1, d * ctx)
    out_t = _tc_mlp_t(flat_t, w1p, b1, W2.T, b2)
    return out_t.T
